# baseline (speedup 1.0000x reference)
import functools

import jax
import jax.numpy as jnp
from jax import lax
from jax.experimental import pallas as pl
from jax.experimental.pallas import tpu as pltpu
from jax.experimental.pallas import tpu_sc as plsc

N = 10000
E = 320000
D = 128
H = 64
S = 3
G = 16
F = S * H
FH = F // 2
W2 = 112
CH = 80
NC = 2
NS = 16
EPT = E // NS
CHUNKS = EPT // CH
RPT_OFF = 624
RPT_SZ = 640


def _dot(a, b):
    return lax.dot_general(a, b, (((1,), (0,)), ((), ())),
                           preferred_element_type=jnp.float32)


def _dot_hp(a, b):
    return lax.dot_general(a, b, (((1,), (0,)), ((), ())),
                           precision=lax.Precision.HIGHEST,
                           preferred_element_type=jnp.float32)


def _tc_pre_body(x_ref, wenc_ref, benc_ref, wcat_ref, pos16_ref, bm_ref,
                 h_ref, td0_ref, td1_ref, ts0_ref, ts1_ref):
    h = jnp.maximum(_dot(x_ref[...], wenc_ref[...]) + benc_ref[...], 0.0)
    h_ref[...] = h
    ab = _dot(h, wcat_ref[...])
    pos16 = pos16_ref[...]
    bm = bm_ref[...]
    td0_ref[...] = jnp.concatenate([ab[:, :FH] + bm[:, :FH], pos16], axis=1)
    td1_ref[...] = jnp.concatenate([ab[:, FH:F] + bm[:, FH:], pos16], axis=1)
    ts0_ref[...] = jnp.concatenate([ab[:, F:F + FH], pos16], axis=1)
    ts1_ref[...] = jnp.concatenate([ab[:, F + FH:], pos16], axis=1)


def _tc_post_body(h_ref, ag_ref, batcht_ref, wu_ref, bu_ref, wp_ref, bp_ref,
                  wg1_ref, bg1_ref, wg2_ref, bg2_ref,
                  wf1_ref, bf1_ref, wf2_ref, bf2_ref, geom_ref, func_ref):
    h = h_ref[...][:, :H]
    ag = jnp.concatenate([ag_ref[0][:, :FH], ag_ref[1][:, :FH]], axis=1)
    cnt = jnp.maximum(ag_ref[0][:, FH:FH + 1], 1.0)
    oht = (lax.broadcasted_iota(jnp.int32, (G, N), 0)
           == batcht_ref[...]).astype(jnp.float32)
    bcnt = jnp.maximum(_dot_hp(oht, jnp.ones((N, 1), jnp.float32)), 1.0)
    feats = []
    for s in range(S):
        aggr_s = ag[:, s * H:(s + 1) * H] / cnt
        ui = jnp.concatenate([h, aggr_s], axis=1)
        hs = jnp.maximum(_dot(ui, wu_ref[s]) + bu_ref[s], 0.0)
        pooled = _dot_hp(oht, hs) / bcnt
        pooled = jnp.maximum(_dot(pooled, wp_ref[s]) + bp_ref[s], 0.0)
        feats.append(pooled)
    msf = jnp.concatenate(feats, axis=1)
    geom_ref[...] = _dot(jnp.maximum(_dot(msf, wg1_ref[...]) + bg1_ref[...],
                                     0.0), wg2_ref[...]) + bg2_ref[...]
    func_ref[...] = _dot(jnp.maximum(_dot(msf, wf1_ref[...]) + bf1_ref[...],
                                     0.0), wf2_ref[...]) + bf2_ref[...]


def _edge_stream(tdst, tsrc, col_hbm, row_hbm, aggr, bufs, wb, eab, s):
    cnt_col = jnp.where(lax.iota(jnp.int32, 16) == 0,
                        jnp.full((16,), 1.0, jnp.float32),
                        jnp.zeros((16,), jnp.float32))
    wbs = [wb[pl.ds(16 * j, 16)] for j in range(FH // 16)]
    base_s = s * EPT

    def load_idx(k, bs):
        base = jnp.where(k < CHUNKS, base_s + k * CH, base_s)
        pltpu.sync_copy(col_hbm.at[pl.ds(base, CH)], bs.colv)
        pltpu.sync_copy(row_hbm.at[pl.ds(base, CH)], bs.rowv)

    def start_gather(bs):
        pltpu.async_copy(tdst.at[bs.colv], bs.dstb, bs.gsem)
        pltpu.async_copy(tsrc.at[bs.rowv], bs.srcb, bs.gsem)

    def wait_gather(bs):
        pltpu.make_async_copy(tdst.at[bs.colv], bs.dstb, bs.gsem).wait()
        pltpu.make_async_copy(tsrc.at[bs.rowv], bs.srcb, bs.gsem).wait()

    def compute(bs):
        dstb, srcb, msgb = bs.dstb, bs.srcb, bs.msgb
        for t in range(CH // 16):
            ev = lax.iota(jnp.int32, 16) + jnp.int32(t * 16)
            d2 = jnp.zeros((16,), jnp.float32)
            for p in range(3):
                cidx = jnp.full((16,), FH + p, jnp.int32)
                dlt = (plsc.load_gather(dstb, [ev, cidx])
                       - plsc.load_gather(srcb, [ev, cidx]))
                d2 = d2 + dlt * dlt
            di = plsc.bitcast(d2, jnp.int32)
            y = plsc.bitcast(jnp.int32(0x5F3759DF) - (di >> 1), jnp.float32)
            for _ in range(3):
                y = y * (1.5 - 0.5 * d2 * y * y)
            ea = jnp.where(d2 > 0.0, d2 * y, 0.0)
            ei = plsc.bitcast(ea, jnp.int32)
            ei = (ei + jnp.int32(0x7FFF) + ((ei >> 16) & jnp.int32(1))) \
                & jnp.int32(-65536)
            eab[pl.ds(t * 16, 16)] = plsc.bitcast(ei, jnp.float32)

        def edge_body(e, carry):
            ea = jnp.broadcast_to(eab[pl.ds(e, 16)][0], (16,))
            for j in range(FH // 16):
                off = j * 16
                m = (dstb[e, pl.ds(off, 16)] + srcb[e, pl.ds(off, 16)]
                     + ea * wbs[j])
                msgb[e, pl.ds(off, 16)] = jnp.maximum(m, 0.0)
            msgb[e, pl.ds(FH, 16)] = cnt_col
            return carry

        lax.fori_loop(0, CH, edge_body, 0)

    def start_scatter(bs):
        pltpu.async_copy(bs.msgb, aggr.at[bs.colv], bs.ssem, add=True)

    def wait_scatter(bs):
        pltpu.make_async_copy(bs.msgb, aggr.at[bs.colv], bs.ssem).wait()

    bA, bB = bufs
    load_idx(0, bA)
    start_gather(bA)

    def body(k2, carry):
        c0 = 2 * k2

        @pl.when(k2 > 0)
        def _():
            wait_scatter(bB)
        load_idx(c0 + 1, bB)
        start_gather(bB)
        wait_gather(bA)
        compute(bA)
        start_scatter(bA)
        wait_scatter(bA)
        load_idx(c0 + 2, bA)
        start_gather(bA)
        wait_gather(bB)
        compute(bB)
        start_scatter(bB)
        return carry

    lax.fori_loop(0, CHUNKS // 2, body, 0)
    wait_gather(bA)
    wait_scatter(bB)


class _BufSet:
    def __init__(self, colv, rowv, dstb, srcb, msgb, gsem, ssem):
        self.colv, self.rowv = colv, rowv
        self.dstb, self.srcb, self.msgb = dstb, srcb, msgb
        self.gsem, self.ssem = gsem, ssem


def _sc_edge_body(td0, td1, ts0, ts1, col_hbm, row_hbm, wvec_hbm,
                  zeros_hbm, out, aggr,
                  colvA, rowvA, dstbA, srcbA, msgbA,
                  colvB, rowvB, dstbB, srcbB, msgbB,
                  wb, eab, gsemA, ssemA, gsemB, ssemB):
    c = lax.axis_index("c")
    s = lax.axis_index("s")
    pltpu.sync_copy(zeros_hbm.at[pl.ds(s * RPT_OFF, RPT_SZ)],
                    aggr.at[pl.ds(s * RPT_OFF, RPT_SZ)])
    pltpu.sync_copy(wvec_hbm.at[c], wb)
    plsc.subcore_barrier()

    bufs = (_BufSet(colvA, rowvA, dstbA, srcbA, msgbA, gsemA, ssemA),
            _BufSet(colvB, rowvB, dstbB, srcbB, msgbB, gsemB, ssemB))

    @pl.when(c == 0)
    def _():
        _edge_stream(td0, ts0, col_hbm, row_hbm, aggr, bufs, wb, eab, s)

    @pl.when(c == 1)
    def _():
        _edge_stream(td1, ts1, col_hbm, row_hbm, aggr, bufs, wb, eab, s)

    plsc.subcore_barrier()
    pltpu.sync_copy(aggr.at[pl.ds(s * RPT_OFF, RPT_SZ)],
                    out.at[c, pl.ds(s * RPT_OFF, RPT_SZ)])


@functools.cache
def _get_sc_edge():
    mesh = plsc.VectorSubcoreMesh(core_axis_name="c", subcore_axis_name="s",
                                  num_cores=NC, num_subcores=NS)
    return pl.kernel(
        _sc_edge_body,
        out_type=jax.ShapeDtypeStruct((NC, N, W2), jnp.float32),
        mesh=mesh,
        compiler_params=pltpu.CompilerParams(needs_layout_passes=False,
                                             use_tc_tiling_on_sc=False),
        scratch_types=[
            pltpu.VMEM_SHARED((N, W2), jnp.float32),
            pltpu.VMEM((CH,), jnp.int32),
            pltpu.VMEM((CH,), jnp.int32),
            pltpu.VMEM((CH, W2), jnp.float32),
            pltpu.VMEM((CH, W2), jnp.float32),
            pltpu.VMEM((CH, W2), jnp.float32),
            pltpu.VMEM((CH,), jnp.int32),
            pltpu.VMEM((CH,), jnp.int32),
            pltpu.VMEM((CH, W2), jnp.float32),
            pltpu.VMEM((CH, W2), jnp.float32),
            pltpu.VMEM((CH, W2), jnp.float32),
            pltpu.VMEM((W2,), jnp.float32),
            pltpu.VMEM((CH + 16,), jnp.float32),
            pltpu.SemaphoreType.DMA,
            pltpu.SemaphoreType.DMA,
            pltpu.SemaphoreType.DMA,
            pltpu.SemaphoreType.DMA,
        ],
    )


_TC_PARAMS = pltpu.CompilerParams(vmem_limit_bytes=110 * 1024 * 1024)

_tc_pre = pl.pallas_call(
    _tc_pre_body,
    out_shape=[jax.ShapeDtypeStruct((N, D), jnp.float32)]
    + [jax.ShapeDtypeStruct((N, W2), jnp.float32)] * 4,
    compiler_params=_TC_PARAMS,
)

_tc_post = pl.pallas_call(
    _tc_post_body,
    out_shape=[jax.ShapeDtypeStruct((G, 1), jnp.float32),
               jax.ShapeDtypeStruct((G, 1), jnp.float32)],
    compiler_params=_TC_PARAMS,
)


def kernel(x, pos, batch, edge_index, W_enc, b_enc,
           Wm0, bm0, Wu0, bu0, Wp0, bp0,
           Wm1, bm1, Wu1, bu1, Wp1, bp1,
           Wm2, bm2, Wu2, bu2, Wp2, bp2,
           Wg1, bg1, Wg2, bg2, Wf1, bf1, Wf2, bf2):
    f32 = jnp.float32
    row = edge_index[0]
    col = edge_index[1]

    wenc_p = jnp.concatenate([W_enc, jnp.zeros((D, D - H), f32)], axis=1)
    benc_p = jnp.concatenate([b_enc, jnp.zeros((D - H,), f32)])[None, :]
    wcat = jnp.concatenate(
        [Wm0[:H], Wm1[:H], Wm2[:H], Wm0[H:2 * H], Wm1[H:2 * H], Wm2[H:2 * H]],
        axis=1)
    wcat = jnp.concatenate([wcat, jnp.zeros((D - H, 2 * F), f32)], axis=0)
    wvec = jnp.concatenate([Wm0[2 * H], Wm1[2 * H], Wm2[2 * H]])
    wvec = wvec.astype(jnp.bfloat16).astype(f32)
    bvec = jnp.concatenate([bm0, bm1, bm2])[None, :]
    pad16 = jnp.zeros((W2 - FH,), f32)
    wvec2 = jnp.stack([jnp.concatenate([wvec[:FH], pad16]),
                       jnp.concatenate([wvec[FH:], pad16])])
    pos16 = jnp.concatenate([pos, jnp.zeros((N, 16 - 3), f32)], axis=1)
    zeros_tab = jnp.zeros((N, W2), f32)

    h_pad, td0, td1, ts0, ts1 = _tc_pre(x, wenc_p, benc_p, wcat, pos16, bvec)
    aggr2 = _get_sc_edge()(td0, td1, ts0, ts1, col, row, wvec2, zeros_tab)

    wu = jnp.stack([Wu0, Wu1, Wu2])
    bu = jnp.stack([bu0, bu1, bu2])[:, None, :]
    wp = jnp.stack([Wp0, Wp1, Wp2])
    bp = jnp.stack([bp0, bp1, bp2])[:, None, :]
    geom, func = _tc_post(h_pad, aggr2, batch[None, :].astype(jnp.int32),
                          wu, bu, wp, bp,
                          Wg1, bg1[None, :], Wg2, bg2[None, :],
                          Wf1, bf1[None, :], Wf2, bf2[None, :])
    return (geom, func)


# speedup vs baseline: 1.3893x; 1.3893x over previous
import functools

import jax
import jax.numpy as jnp
from jax import lax
from jax.experimental import pallas as pl
from jax.experimental.pallas import tpu as pltpu
from jax.experimental.pallas import tpu_sc as plsc

N = 10000
E = 320000
D = 128
H = 64
S = 3
G = 16
F = S * H
FH = F // 2
W2 = 112
CH = 80
NC = 2
NS = 16
EPT = E // NS
CHUNKS = EPT // CH
SUP = 25
NSUP = CHUNKS // SUP
IRT = EPT // CH
RPT_OFF = 624
RPT_SZ = 640


def _dot(a, b):
    return lax.dot_general(a, b, (((1,), (0,)), ((), ())),
                           preferred_element_type=jnp.float32)


def _dot_hp(a, b):
    return lax.dot_general(a, b, (((1,), (0,)), ((), ())),
                           precision=lax.Precision.HIGHEST,
                           preferred_element_type=jnp.float32)


def _tc_pre_body(x_ref, wenc_ref, benc_ref, wcat_ref, pos16_ref, bm_ref,
                 h_ref, td0_ref, td1_ref, ts0_ref, ts1_ref):
    h = jnp.maximum(_dot(x_ref[...], wenc_ref[...]) + benc_ref[...], 0.0)
    h_ref[...] = h
    ab = _dot(h, wcat_ref[...])
    pos16 = pos16_ref[...]
    bm = bm_ref[...]
    td0_ref[...] = jnp.concatenate([ab[:, :FH] + bm[:, :FH], pos16], axis=1)
    td1_ref[...] = jnp.concatenate([ab[:, FH:F] + bm[:, FH:], pos16], axis=1)
    ts0_ref[...] = jnp.concatenate([ab[:, F:F + FH], pos16], axis=1)
    ts1_ref[...] = jnp.concatenate([ab[:, F + FH:], pos16], axis=1)


def _tc_post_body(h_ref, ag_ref, batcht_ref, wu_ref, bu_ref, wp_ref, bp_ref,
                  wg1_ref, bg1_ref, wg2_ref, bg2_ref,
                  wf1_ref, bf1_ref, wf2_ref, bf2_ref, geom_ref, func_ref):
    h = h_ref[...][:, :H]
    ag = jnp.concatenate([ag_ref[0][:, :FH], ag_ref[1][:, :FH]], axis=1)
    cnt = jnp.maximum(ag_ref[0][:, FH:FH + 1], 1.0)
    oht = (lax.broadcasted_iota(jnp.int32, (G, N), 0)
           == batcht_ref[...]).astype(jnp.float32)
    bcnt = jnp.maximum(_dot_hp(oht, jnp.ones((N, 1), jnp.float32)), 1.0)
    feats = []
    for s in range(S):
        aggr_s = ag[:, s * H:(s + 1) * H] / cnt
        ui = jnp.concatenate([h, aggr_s], axis=1)
        hs = jnp.maximum(_dot(ui, wu_ref[s]) + bu_ref[s], 0.0)
        pooled = _dot_hp(oht, hs) / bcnt
        pooled = jnp.maximum(_dot(pooled, wp_ref[s]) + bp_ref[s], 0.0)
        feats.append(pooled)
    msf = jnp.concatenate(feats, axis=1)
    geom_ref[...] = _dot(jnp.maximum(_dot(msf, wg1_ref[...]) + bg1_ref[...],
                                     0.0), wg2_ref[...]) + bg2_ref[...]
    func_ref[...] = _dot(jnp.maximum(_dot(msf, wf1_ref[...]) + bf1_ref[...],
                                     0.0), wf2_ref[...]) + bf2_ref[...]


def _edge_stream(tdst, tsrc, col2_hbm, row2_hbm, aggr, bufs, colsb, rowsb,
                 wb, s):
    cnt_col = jnp.where(lax.iota(jnp.int32, 16) == 0,
                        jnp.full((16,), 1.0, jnp.float32),
                        jnp.zeros((16,), jnp.float32))
    wbs = [wb[pl.ds(16 * j, 16)] for j in range(FH // 16)]
    srow0 = s * IRT

    def start_gather(bs, i):
        pltpu.async_copy(tdst.at[colsb.at[i]], bs.dstb, bs.gsem)
        pltpu.async_copy(tsrc.at[rowsb.at[i]], bs.srcb, bs.gsem)

    def wait_gather(bs, i):
        pltpu.make_async_copy(tdst.at[colsb.at[i]], bs.dstb, bs.gsem).wait()
        pltpu.make_async_copy(tsrc.at[rowsb.at[i]], bs.srcb, bs.gsem).wait()

    def compute(bs):
        dstb, srcb, msgb = bs.dstb, bs.srcb, bs.msgb

        def edge_body(e, carry):
            diff = dstb[e, pl.ds(FH, 16)] - srcb[e, pl.ds(FH, 16)]
            sq = diff * diff
            d2 = jnp.broadcast_to(sq[0] + sq[1] + sq[2], (16,))
            di = plsc.bitcast(d2, jnp.int32)
            y = plsc.bitcast(jnp.int32(0x5F3759DF) - (di >> 1), jnp.float32)
            for _ in range(3):
                y = y * (1.5 - 0.5 * d2 * y * y)
            ea = jnp.where(d2 > 0.0, d2 * y, 0.0)
            ei = plsc.bitcast(ea, jnp.int32)
            ei = (ei + jnp.int32(0x7FFF) + ((ei >> 16) & jnp.int32(1))) \
                & jnp.int32(-65536)
            ea = plsc.bitcast(ei, jnp.float32)
            for j in range(FH // 16):
                off = j * 16
                m = (dstb[e, pl.ds(off, 16)] + srcb[e, pl.ds(off, 16)]
                     + ea * wbs[j])
                msgb[e, pl.ds(off, 16)] = jnp.maximum(m, 0.0)
            msgb[e, pl.ds(FH, 16)] = cnt_col
            return carry

        lax.fori_loop(0, CH, edge_body, 0)

    def start_scatter(bs, i):
        pltpu.async_copy(bs.msgb, aggr.at[colsb.at[i]], bs.ssem, add=True)

    def wait_scatter(bs, i):
        pltpu.make_async_copy(bs.msgb, aggr.at[colsb.at[i]], bs.ssem).wait()

    bA, bB = bufs

    def super_body(sk, carry):
        srow = srow0 + sk * SUP
        pltpu.sync_copy(col2_hbm.at[pl.ds(srow, SUP)], colsb)
        pltpu.sync_copy(row2_hbm.at[pl.ds(srow, SUP)], rowsb)
        start_gather(bA, 0)

        def pair_body(k2, carry2):
            i0 = 2 * k2

            @pl.when(k2 > 0)
            def _():
                wait_scatter(bB, i0 - 1)
            start_gather(bB, i0 + 1)
            wait_gather(bA, i0)
            compute(bA)
            start_scatter(bA, i0)
            wait_scatter(bA, i0)
            start_gather(bA, i0 + 2)
            wait_gather(bB, i0 + 1)
            compute(bB)
            start_scatter(bB, i0 + 1)
            return carry2

        lax.fori_loop(0, SUP // 2, pair_body, 0)
        wait_scatter(bB, SUP - 2)
        wait_gather(bA, SUP - 1)
        compute(bA)
        start_scatter(bA, SUP - 1)
        wait_scatter(bA, SUP - 1)
        return carry

    lax.fori_loop(0, NSUP, super_body, 0)


class _BufSet:
    def __init__(self, dstb, srcb, msgb, gsem, ssem):
        self.dstb, self.srcb, self.msgb = dstb, srcb, msgb
        self.gsem, self.ssem = gsem, ssem


def _sc_edge_body(td0, td1, ts0, ts1, col_hbm, row_hbm, wvec_hbm,
                  zeros_hbm, out, aggr, colsb, rowsb,
                  dstbA, srcbA, msgbA, dstbB, srcbB, msgbB,
                  wb, gsemA, ssemA, gsemB, ssemB):
    c = lax.axis_index("c")
    s = lax.axis_index("s")
    pltpu.sync_copy(zeros_hbm.at[pl.ds(s * RPT_OFF, RPT_SZ)],
                    aggr.at[pl.ds(s * RPT_OFF, RPT_SZ)])
    pltpu.sync_copy(wvec_hbm.at[c], wb)
    plsc.subcore_barrier()

    bufs = (_BufSet(dstbA, srcbA, msgbA, gsemA, ssemA),
            _BufSet(dstbB, srcbB, msgbB, gsemB, ssemB))

    @pl.when(c == 0)
    def _():
        _edge_stream(td0, ts0, col_hbm, row_hbm, aggr, bufs, colsb, rowsb,
                     wb, s)

    @pl.when(c == 1)
    def _():
        _edge_stream(td1, ts1, col_hbm, row_hbm, aggr, bufs, colsb, rowsb,
                     wb, s)

    plsc.subcore_barrier()
    pltpu.sync_copy(aggr.at[pl.ds(s * RPT_OFF, RPT_SZ)],
                    out.at[c, pl.ds(s * RPT_OFF, RPT_SZ)])


@functools.cache
def _get_sc_edge():
    mesh = plsc.VectorSubcoreMesh(core_axis_name="c", subcore_axis_name="s",
                                  num_cores=NC, num_subcores=NS)
    return pl.kernel(
        _sc_edge_body,
        out_type=jax.ShapeDtypeStruct((NC, N, W2), jnp.float32),
        mesh=mesh,
        compiler_params=pltpu.CompilerParams(needs_layout_passes=False,
                                             use_tc_tiling_on_sc=False),
        scratch_types=[
            pltpu.VMEM_SHARED((N, W2), jnp.float32),
            pltpu.VMEM((SUP, CH), jnp.int32),
            pltpu.VMEM((SUP, CH), jnp.int32),
            pltpu.VMEM((CH, W2), jnp.float32),
            pltpu.VMEM((CH, W2), jnp.float32),
            pltpu.VMEM((CH, W2), jnp.float32),
            pltpu.VMEM((CH, W2), jnp.float32),
            pltpu.VMEM((CH, W2), jnp.float32),
            pltpu.VMEM((CH, W2), jnp.float32),
            pltpu.VMEM((W2,), jnp.float32),
            pltpu.SemaphoreType.DMA,
            pltpu.SemaphoreType.DMA,
            pltpu.SemaphoreType.DMA,
            pltpu.SemaphoreType.DMA,
        ],
    )


_TC_PARAMS = pltpu.CompilerParams(vmem_limit_bytes=110 * 1024 * 1024)

_tc_pre = pl.pallas_call(
    _tc_pre_body,
    out_shape=[jax.ShapeDtypeStruct((N, D), jnp.float32)]
    + [jax.ShapeDtypeStruct((N, W2), jnp.float32)] * 4,
    compiler_params=_TC_PARAMS,
)

_tc_post = pl.pallas_call(
    _tc_post_body,
    out_shape=[jax.ShapeDtypeStruct((G, 1), jnp.float32),
               jax.ShapeDtypeStruct((G, 1), jnp.float32)],
    compiler_params=_TC_PARAMS,
)


def kernel(x, pos, batch, edge_index, W_enc, b_enc,
           Wm0, bm0, Wu0, bu0, Wp0, bp0,
           Wm1, bm1, Wu1, bu1, Wp1, bp1,
           Wm2, bm2, Wu2, bu2, Wp2, bp2,
           Wg1, bg1, Wg2, bg2, Wf1, bf1, Wf2, bf2):
    f32 = jnp.float32
    row = edge_index[0]
    col = edge_index[1]

    wenc_p = jnp.concatenate([W_enc, jnp.zeros((D, D - H), f32)], axis=1)
    benc_p = jnp.concatenate([b_enc, jnp.zeros((D - H,), f32)])[None, :]
    wcat = jnp.concatenate(
        [Wm0[:H], Wm1[:H], Wm2[:H], Wm0[H:2 * H], Wm1[H:2 * H], Wm2[H:2 * H]],
        axis=1)
    wcat = jnp.concatenate([wcat, jnp.zeros((D - H, 2 * F), f32)], axis=0)
    wvec = jnp.concatenate([Wm0[2 * H], Wm1[2 * H], Wm2[2 * H]])
    wvec = wvec.astype(jnp.bfloat16).astype(f32)
    bvec = jnp.concatenate([bm0, bm1, bm2])[None, :]
    pad16 = jnp.zeros((W2 - FH,), f32)
    wvec2 = jnp.stack([jnp.concatenate([wvec[:FH], pad16]),
                       jnp.concatenate([wvec[FH:], pad16])])
    pos16 = jnp.concatenate([pos, jnp.zeros((N, 16 - 3), f32)], axis=1)
    zeros_tab = jnp.zeros((N, W2), f32)

    h_pad, td0, td1, ts0, ts1 = _tc_pre(x, wenc_p, benc_p, wcat, pos16, bvec)
    col2 = col.reshape(E // CH, CH)
    row2 = row.reshape(E // CH, CH)
    aggr2 = _get_sc_edge()(td0, td1, ts0, ts1, col2, row2, wvec2, zeros_tab)

    wu = jnp.stack([Wu0, Wu1, Wu2])
    bu = jnp.stack([bu0, bu1, bu2])[:, None, :]
    wp = jnp.stack([Wp0, Wp1, Wp2])
    bp = jnp.stack([bp0, bp1, bp2])[:, None, :]
    geom, func = _tc_post(h_pad, aggr2, batch[None, :].astype(jnp.int32),
                          wu, bu, wp, bp,
                          Wg1, bg1[None, :], Wg2, bg2[None, :],
                          Wf1, bf1[None, :], Wf2, bf2[None, :])
    return (geom, func)


# speedup vs baseline: 1.4102x; 1.0150x over previous
import functools

import jax
import jax.numpy as jnp
from jax import lax
from jax.experimental import pallas as pl
from jax.experimental.pallas import tpu as pltpu
from jax.experimental.pallas import tpu_sc as plsc

N = 10000
E = 320000
D = 128
H = 64
S = 3
G = 16
F = S * H
FH = F // 2
W2 = 112
CH = 80
NC = 2
NS = 16
EPT = E // NS
CHUNKS = EPT // CH
SUP = 25
NSUP = CHUNKS // SUP
IRT = EPT // CH
RPT_OFF = 624
RPT_SZ = 640


def _dot(a, b):
    return lax.dot_general(a, b, (((1,), (0,)), ((), ())),
                           preferred_element_type=jnp.float32)


def _dot_hp(a, b):
    return lax.dot_general(a, b, (((1,), (0,)), ((), ())),
                           precision=lax.Precision.HIGHEST,
                           preferred_element_type=jnp.float32)


def _tc_pre_body(x_ref, wenc_ref, benc_ref, wcat_ref, pos16_ref, bm_ref,
                 h_ref, td0_ref, td1_ref, ts0_ref, ts1_ref):
    h = jnp.maximum(_dot(x_ref[...], wenc_ref[...]) + benc_ref[...], 0.0)
    h_ref[...] = h
    ab = _dot(h, wcat_ref[...])
    pos16 = pos16_ref[...]
    bm = bm_ref[...]
    td0_ref[...] = jnp.concatenate([ab[:, :FH] + bm[:, :FH], pos16], axis=1)
    td1_ref[...] = jnp.concatenate([ab[:, FH:F] + bm[:, FH:], pos16], axis=1)
    ts0_ref[...] = jnp.concatenate([ab[:, F:F + FH], pos16], axis=1)
    ts1_ref[...] = jnp.concatenate([ab[:, F + FH:], pos16], axis=1)


def _tc_post_body(h_ref, ag_ref, batcht_ref, wu_ref, bu_ref, wp_ref, bp_ref,
                  wg1_ref, bg1_ref, wg2_ref, bg2_ref,
                  wf1_ref, bf1_ref, wf2_ref, bf2_ref, geom_ref, func_ref):
    h = h_ref[...][:, :H]
    ag = jnp.concatenate([ag_ref[0][:, :FH], ag_ref[1][:, :FH]], axis=1)
    cnt = jnp.maximum(ag_ref[0][:, FH:FH + 1], 1.0)
    oht = (lax.broadcasted_iota(jnp.int32, (G, N), 0)
           == batcht_ref[...]).astype(jnp.float32)
    bcnt = jnp.maximum(_dot_hp(oht, jnp.ones((N, 1), jnp.float32)), 1.0)
    feats = []
    for s in range(S):
        aggr_s = ag[:, s * H:(s + 1) * H] / cnt
        ui = jnp.concatenate([h, aggr_s], axis=1)
        hs = jnp.maximum(_dot(ui, wu_ref[s]) + bu_ref[s], 0.0)
        pooled = _dot_hp(oht, hs) / bcnt
        pooled = jnp.maximum(_dot(pooled, wp_ref[s]) + bp_ref[s], 0.0)
        feats.append(pooled)
    msf = jnp.concatenate(feats, axis=1)
    geom_ref[...] = _dot(jnp.maximum(_dot(msf, wg1_ref[...]) + bg1_ref[...],
                                     0.0), wg2_ref[...]) + bg2_ref[...]
    func_ref[...] = _dot(jnp.maximum(_dot(msf, wf1_ref[...]) + bf1_ref[...],
                                     0.0), wf2_ref[...]) + bf2_ref[...]


def _edge_stream(tdst, tsrc, idx2_hbm, aggr, bufs, idxb, wb, s):
    cnt_col = jnp.where(lax.iota(jnp.int32, 16) == 0,
                        jnp.full((16,), 1.0, jnp.float32),
                        jnp.zeros((16,), jnp.float32))
    wbs = [wb[pl.ds(16 * j, 16)] for j in range(FH // 16)]
    srow0 = s * IRT

    def start_gather(bs, i):
        pltpu.async_copy(tdst.at[idxb.at[i, 0]], bs.dstb, bs.gsem)
        pltpu.async_copy(tsrc.at[idxb.at[i, 1]], bs.srcb, bs.gsem)

    def wait_gather(bs, i):
        pltpu.make_async_copy(tdst.at[idxb.at[i, 0]], bs.dstb,
                              bs.gsem).wait()
        pltpu.make_async_copy(tsrc.at[idxb.at[i, 1]], bs.srcb,
                              bs.gsem).wait()

    def compute(bs):
        dstb, srcb, msgb = bs.dstb, bs.srcb, bs.msgb

        def edge_body(e, carry):
            diff = dstb[e, pl.ds(FH, 16)] - srcb[e, pl.ds(FH, 16)]
            sq = diff * diff
            d2 = jnp.broadcast_to(sq[0] + sq[1] + sq[2], (16,))
            di = plsc.bitcast(d2, jnp.int32)
            y = plsc.bitcast(jnp.int32(0x5F3759DF) - (di >> 1), jnp.float32)
            for _ in range(3):
                y = y * (1.5 - 0.5 * d2 * y * y)
            ea = jnp.where(d2 > 0.0, d2 * y, 0.0)
            ei = plsc.bitcast(ea, jnp.int32)
            ei = (ei + jnp.int32(0x7FFF) + ((ei >> 16) & jnp.int32(1))) \
                & jnp.int32(-65536)
            ea = plsc.bitcast(ei, jnp.float32)
            for j in range(FH // 16):
                off = j * 16
                m = (dstb[e, pl.ds(off, 16)] + srcb[e, pl.ds(off, 16)]
                     + ea * wbs[j])
                msgb[e, pl.ds(off, 16)] = jnp.maximum(m, 0.0)
            msgb[e, pl.ds(FH, 16)] = cnt_col
            return carry

        lax.fori_loop(0, CH, edge_body, 0)

    def start_scatter(bs, i):
        pltpu.async_copy(bs.msgb, aggr.at[idxb.at[i, 0]], bs.ssem, add=True)

    def wait_scatter(bs, i):
        pltpu.make_async_copy(bs.msgb, aggr.at[idxb.at[i, 0]],
                              bs.ssem).wait()

    bA, bB = bufs

    def super_body(sk, carry):
        srow = srow0 + sk * SUP
        pltpu.sync_copy(idx2_hbm.at[pl.ds(srow, SUP)], idxb)
        start_gather(bA, 0)

        def pair_body(k2, carry2):
            i0 = 2 * k2

            @pl.when(k2 > 0)
            def _():
                wait_scatter(bB, i0 - 1)
            start_gather(bB, i0 + 1)

            @pl.when(k2 > 0)
            def _():
                wait_scatter(bA, i0 - 2)
            wait_gather(bA, i0)
            compute(bA)
            start_scatter(bA, i0)
            start_gather(bA, i0 + 2)
            wait_gather(bB, i0 + 1)
            compute(bB)
            start_scatter(bB, i0 + 1)
            return carry2

        lax.fori_loop(0, SUP // 2, pair_body, 0)
        wait_scatter(bA, SUP - 3)
        wait_gather(bA, SUP - 1)
        compute(bA)
        start_scatter(bA, SUP - 1)
        wait_scatter(bB, SUP - 2)
        wait_scatter(bA, SUP - 1)
        return carry

    lax.fori_loop(0, NSUP, super_body, 0)


class _BufSet:
    def __init__(self, dstb, srcb, msgb, gsem, ssem):
        self.dstb, self.srcb, self.msgb = dstb, srcb, msgb
        self.gsem, self.ssem = gsem, ssem


def _sc_edge_body(td0, td1, ts0, ts1, idx2_hbm, wvec_hbm,
                  zeros_hbm, out, aggr, idxb,
                  dstbA, srcbA, msgbA, dstbB, srcbB, msgbB,
                  wb, gsemA, ssemA, gsemB, ssemB):
    c = lax.axis_index("c")
    s = lax.axis_index("s")
    pltpu.sync_copy(zeros_hbm.at[pl.ds(s * RPT_OFF, RPT_SZ)],
                    aggr.at[pl.ds(s * RPT_OFF, RPT_SZ)])
    pltpu.sync_copy(wvec_hbm.at[c], wb)
    plsc.subcore_barrier()

    bufs = (_BufSet(dstbA, srcbA, msgbA, gsemA, ssemA),
            _BufSet(dstbB, srcbB, msgbB, gsemB, ssemB))

    @pl.when(c == 0)
    def _():
        _edge_stream(td0, ts0, idx2_hbm, aggr, bufs, idxb, wb, s)

    @pl.when(c == 1)
    def _():
        _edge_stream(td1, ts1, idx2_hbm, aggr, bufs, idxb, wb, s)

    plsc.subcore_barrier()
    pltpu.sync_copy(aggr.at[pl.ds(s * RPT_OFF, RPT_SZ)],
                    out.at[c, pl.ds(s * RPT_OFF, RPT_SZ)])


@functools.cache
def _get_sc_edge():
    mesh = plsc.VectorSubcoreMesh(core_axis_name="c", subcore_axis_name="s",
                                  num_cores=NC, num_subcores=NS)
    return pl.kernel(
        _sc_edge_body,
        out_type=jax.ShapeDtypeStruct((NC, N, W2), jnp.float32),
        mesh=mesh,
        compiler_params=pltpu.CompilerParams(needs_layout_passes=False,
                                             use_tc_tiling_on_sc=False),
        scratch_types=[
            pltpu.VMEM_SHARED((N, W2), jnp.float32),
            pltpu.VMEM((SUP, 2, CH), jnp.int32),
            pltpu.VMEM((CH, W2), jnp.float32),
            pltpu.VMEM((CH, W2), jnp.float32),
            pltpu.VMEM((CH, W2), jnp.float32),
            pltpu.VMEM((CH, W2), jnp.float32),
            pltpu.VMEM((CH, W2), jnp.float32),
            pltpu.VMEM((CH, W2), jnp.float32),
            pltpu.VMEM((W2,), jnp.float32),
            pltpu.SemaphoreType.DMA,
            pltpu.SemaphoreType.DMA,
            pltpu.SemaphoreType.DMA,
            pltpu.SemaphoreType.DMA,
        ],
    )


_TC_PARAMS = pltpu.CompilerParams(vmem_limit_bytes=110 * 1024 * 1024)

_tc_pre = pl.pallas_call(
    _tc_pre_body,
    out_shape=[jax.ShapeDtypeStruct((N, D), jnp.float32)]
    + [jax.ShapeDtypeStruct((N, W2), jnp.float32)] * 4,
    compiler_params=_TC_PARAMS,
)

_tc_post = pl.pallas_call(
    _tc_post_body,
    out_shape=[jax.ShapeDtypeStruct((G, 1), jnp.float32),
               jax.ShapeDtypeStruct((G, 1), jnp.float32)],
    compiler_params=_TC_PARAMS,
)


def kernel(x, pos, batch, edge_index, W_enc, b_enc,
           Wm0, bm0, Wu0, bu0, Wp0, bp0,
           Wm1, bm1, Wu1, bu1, Wp1, bp1,
           Wm2, bm2, Wu2, bu2, Wp2, bp2,
           Wg1, bg1, Wg2, bg2, Wf1, bf1, Wf2, bf2):
    f32 = jnp.float32
    row = edge_index[0]
    col = edge_index[1]

    wenc_p = jnp.concatenate([W_enc, jnp.zeros((D, D - H), f32)], axis=1)
    benc_p = jnp.concatenate([b_enc, jnp.zeros((D - H,), f32)])[None, :]
    wcat = jnp.concatenate(
        [Wm0[:H], Wm1[:H], Wm2[:H], Wm0[H:2 * H], Wm1[H:2 * H], Wm2[H:2 * H]],
        axis=1)
    wcat = jnp.concatenate([wcat, jnp.zeros((D - H, 2 * F), f32)], axis=0)
    wvec = jnp.concatenate([Wm0[2 * H], Wm1[2 * H], Wm2[2 * H]])
    wvec = wvec.astype(jnp.bfloat16).astype(f32)
    bvec = jnp.concatenate([bm0, bm1, bm2])[None, :]
    pad16 = jnp.zeros((W2 - FH,), f32)
    wvec2 = jnp.stack([jnp.concatenate([wvec[:FH], pad16]),
                       jnp.concatenate([wvec[FH:], pad16])])
    pos16 = jnp.concatenate([pos, jnp.zeros((N, 16 - 3), f32)], axis=1)
    zeros_tab = jnp.zeros((N, W2), f32)

    h_pad, td0, td1, ts0, ts1 = _tc_pre(x, wenc_p, benc_p, wcat, pos16, bvec)
    idx2 = jnp.stack([col.reshape(E // CH, CH), row.reshape(E // CH, CH)],
                     axis=1)
    aggr2 = _get_sc_edge()(td0, td1, ts0, ts1, idx2, wvec2, zeros_tab)

    wu = jnp.stack([Wu0, Wu1, Wu2])
    bu = jnp.stack([bu0, bu1, bu2])[:, None, :]
    wp = jnp.stack([Wp0, Wp1, Wp2])
    bp = jnp.stack([bp0, bp1, bp2])[:, None, :]
    geom, func = _tc_post(h_pad, aggr2, batch[None, :].astype(jnp.int32),
                          wu, bu, wp, bp,
                          Wg1, bg1[None, :], Wg2, bg2[None, :],
                          Wf1, bf1[None, :], Wf2, bf2[None, :])
    return (geom, func)


# speedup vs baseline: 1.4255x; 1.0108x over previous
import functools

import jax
import jax.numpy as jnp
from jax import lax
from jax.experimental import pallas as pl
from jax.experimental.pallas import tpu as pltpu
from jax.experimental.pallas import tpu_sc as plsc

N = 10000
E = 320000
D = 128
H = 64
S = 3
G = 16
F = S * H
FH = F // 2
W2 = 112
CH = 80
NC = 2
NS = 16
EPT = E // NS
CHUNKS = EPT // CH
SUP = 25
NSUP = CHUNKS // SUP
IRT = EPT // CH
RPT_OFF = 624
RPT_SZ = 640


def _dot(a, b):
    return lax.dot_general(a, b, (((1,), (0,)), ((), ())),
                           preferred_element_type=jnp.float32)


def _dot_hp(a, b):
    return lax.dot_general(a, b, (((1,), (0,)), ((), ())),
                           precision=lax.Precision.HIGHEST,
                           preferred_element_type=jnp.float32)


def _tc_pre_body(x_ref, wenc_ref, benc_ref, wcat_ref, pos16_ref, bm_ref,
                 h_ref, td0_ref, td1_ref, ts0_ref, ts1_ref):
    h = jnp.maximum(_dot(x_ref[...], wenc_ref[...]) + benc_ref[...], 0.0)
    h_ref[...] = h
    ab = _dot(h, wcat_ref[...])
    pos16 = pos16_ref[...]
    bm = bm_ref[...]
    td0_ref[...] = jnp.concatenate([ab[:, :FH] + bm[:, :FH], pos16], axis=1)
    td1_ref[...] = jnp.concatenate([ab[:, FH:F] + bm[:, FH:], pos16], axis=1)
    ts0_ref[...] = jnp.concatenate([ab[:, F:F + FH], pos16], axis=1)
    ts1_ref[...] = jnp.concatenate([ab[:, F + FH:], pos16], axis=1)


def _tc_post_body(h_ref, ag_ref, batcht_ref, wu_ref, bu_ref, wp_ref, bp_ref,
                  wg1_ref, bg1_ref, wg2_ref, bg2_ref,
                  wf1_ref, bf1_ref, wf2_ref, bf2_ref, geom_ref, func_ref):
    h = h_ref[...][:, :H]
    ag = jnp.concatenate([ag_ref[0][:, :FH], ag_ref[1][:, :FH]], axis=1)
    cnt = jnp.maximum(ag_ref[0][:, FH:FH + 1], 1.0)
    oht = (lax.broadcasted_iota(jnp.int32, (G, N), 0)
           == batcht_ref[...]).astype(jnp.float32)
    bcnt = jnp.maximum(_dot_hp(oht, jnp.ones((N, 1), jnp.float32)), 1.0)
    feats = []
    for s in range(S):
        aggr_s = ag[:, s * H:(s + 1) * H] / cnt
        ui = jnp.concatenate([h, aggr_s], axis=1)
        hs = jnp.maximum(_dot(ui, wu_ref[s]) + bu_ref[s], 0.0)
        pooled = _dot_hp(oht, hs) / bcnt
        pooled = jnp.maximum(_dot(pooled, wp_ref[s]) + bp_ref[s], 0.0)
        feats.append(pooled)
    msf = jnp.concatenate(feats, axis=1)
    geom_ref[...] = _dot(jnp.maximum(_dot(msf, wg1_ref[...]) + bg1_ref[...],
                                     0.0), wg2_ref[...]) + bg2_ref[...]
    func_ref[...] = _dot(jnp.maximum(_dot(msf, wf1_ref[...]) + bf1_ref[...],
                                     0.0), wf2_ref[...]) + bf2_ref[...]


def _edge_stream(tdst, tsrc, idx2_hbm, aggr, bufs, idxb, wb, s):
    cnt_col = jnp.where(lax.iota(jnp.int32, 16) == 0,
                        jnp.full((16,), 1.0, jnp.float32),
                        jnp.zeros((16,), jnp.float32))
    wbs = [wb[pl.ds(16 * j, 16)] for j in range(FH // 16)]
    srow0 = s * IRT

    def start_gather(bs, i):
        pltpu.async_copy(tdst.at[idxb.at[i, 0]], bs.dstb, bs.gsem)
        pltpu.async_copy(tsrc.at[idxb.at[i, 1]], bs.srcb, bs.gsem)

    def wait_gather(bs, i):
        pltpu.make_async_copy(tdst.at[idxb.at[i, 0]], bs.dstb,
                              bs.gsem).wait()
        pltpu.make_async_copy(tsrc.at[idxb.at[i, 1]], bs.srcb,
                              bs.gsem).wait()

    def compute(bs):
        dstb, srcb, msgb = bs.dstb, bs.srcb, bs.msgb

        def edge_body(e, carry):
            diff = dstb[e, pl.ds(FH, 16)] - srcb[e, pl.ds(FH, 16)]
            sq = diff * diff
            d2 = jnp.broadcast_to(sq[0] + sq[1] + sq[2], (16,))
            di = plsc.bitcast(d2, jnp.int32)
            y = plsc.bitcast(jnp.int32(0x5F3759DF) - (di >> 1), jnp.float32)
            for _ in range(3):
                y = y * (1.5 - 0.5 * d2 * y * y)
            ea = d2 * y
            ei = plsc.bitcast(ea, jnp.int32)
            ei = (ei + jnp.int32(0x7FFF) + ((ei >> 16) & jnp.int32(1))) \
                & jnp.int32(-65536)
            ea = plsc.bitcast(ei, jnp.float32)
            for j in range(FH // 16):
                off = j * 16
                m = (dstb[e, pl.ds(off, 16)] + srcb[e, pl.ds(off, 16)]
                     + ea * wbs[j])
                msgb[e, pl.ds(off, 16)] = jnp.maximum(m, 0.0)
            return carry

        lax.fori_loop(0, CH, edge_body, 0)

    def start_scatter(bs, i):
        pltpu.async_copy(bs.msgb, aggr.at[idxb.at[i, 0]], bs.ssem, add=True)

    def wait_scatter(bs, i):
        pltpu.make_async_copy(bs.msgb, aggr.at[idxb.at[i, 0]],
                              bs.ssem).wait()

    bA, bB = bufs

    def fill_cnt(msgb):
        def fb(e, cr):
            msgb[e, pl.ds(FH, 16)] = cnt_col
            return cr
        lax.fori_loop(0, CH, fb, 0)

    fill_cnt(bA.msgb)
    fill_cnt(bB.msgb)

    def super_body(sk, carry):
        srow = srow0 + sk * SUP
        pltpu.sync_copy(idx2_hbm.at[pl.ds(srow, SUP)], idxb)
        start_gather(bA, 0)

        def pair_body(k2, carry2):
            i0 = 2 * k2

            @pl.when(k2 > 0)
            def _():
                wait_scatter(bB, i0 - 1)
            start_gather(bB, i0 + 1)

            @pl.when(k2 > 0)
            def _():
                wait_scatter(bA, i0 - 2)
            wait_gather(bA, i0)
            compute(bA)
            start_scatter(bA, i0)
            start_gather(bA, i0 + 2)
            wait_gather(bB, i0 + 1)
            compute(bB)
            start_scatter(bB, i0 + 1)
            return carry2

        lax.fori_loop(0, SUP // 2, pair_body, 0)
        wait_scatter(bA, SUP - 3)
        wait_gather(bA, SUP - 1)
        compute(bA)
        start_scatter(bA, SUP - 1)
        wait_scatter(bB, SUP - 2)
        wait_scatter(bA, SUP - 1)
        return carry

    lax.fori_loop(0, NSUP, super_body, 0)


class _BufSet:
    def __init__(self, dstb, srcb, msgb, gsem, ssem):
        self.dstb, self.srcb, self.msgb = dstb, srcb, msgb
        self.gsem, self.ssem = gsem, ssem


def _sc_edge_body(td0, td1, ts0, ts1, idx2_hbm, wvec_hbm,
                  zeros_hbm, out, aggr, idxb,
                  dstbA, srcbA, msgbA, dstbB, srcbB, msgbB,
                  wb, gsemA, ssemA, gsemB, ssemB):
    c = lax.axis_index("c")
    s = lax.axis_index("s")
    pltpu.sync_copy(zeros_hbm.at[pl.ds(s * RPT_OFF, RPT_SZ)],
                    aggr.at[pl.ds(s * RPT_OFF, RPT_SZ)])
    pltpu.sync_copy(wvec_hbm.at[c], wb)
    plsc.subcore_barrier()

    bufs = (_BufSet(dstbA, srcbA, msgbA, gsemA, ssemA),
            _BufSet(dstbB, srcbB, msgbB, gsemB, ssemB))

    @pl.when(c == 0)
    def _():
        _edge_stream(td0, ts0, idx2_hbm, aggr, bufs, idxb, wb, s)

    @pl.when(c == 1)
    def _():
        _edge_stream(td1, ts1, idx2_hbm, aggr, bufs, idxb, wb, s)

    plsc.subcore_barrier()
    pltpu.sync_copy(aggr.at[pl.ds(s * RPT_OFF, RPT_SZ)],
                    out.at[c, pl.ds(s * RPT_OFF, RPT_SZ)])


@functools.cache
def _get_sc_edge():
    mesh = plsc.VectorSubcoreMesh(core_axis_name="c", subcore_axis_name="s",
                                  num_cores=NC, num_subcores=NS)
    return pl.kernel(
        _sc_edge_body,
        out_type=jax.ShapeDtypeStruct((NC, N, W2), jnp.float32),
        mesh=mesh,
        compiler_params=pltpu.CompilerParams(needs_layout_passes=False,
                                             use_tc_tiling_on_sc=False),
        scratch_types=[
            pltpu.VMEM_SHARED((N, W2), jnp.float32),
            pltpu.VMEM((SUP, 2, CH), jnp.int32),
            pltpu.VMEM((CH, W2), jnp.float32),
            pltpu.VMEM((CH, W2), jnp.float32),
            pltpu.VMEM((CH, W2), jnp.float32),
            pltpu.VMEM((CH, W2), jnp.float32),
            pltpu.VMEM((CH, W2), jnp.float32),
            pltpu.VMEM((CH, W2), jnp.float32),
            pltpu.VMEM((W2,), jnp.float32),
            pltpu.SemaphoreType.DMA,
            pltpu.SemaphoreType.DMA,
            pltpu.SemaphoreType.DMA,
            pltpu.SemaphoreType.DMA,
        ],
    )


_TC_PARAMS = pltpu.CompilerParams(vmem_limit_bytes=110 * 1024 * 1024)

_tc_pre = pl.pallas_call(
    _tc_pre_body,
    out_shape=[jax.ShapeDtypeStruct((N, D), jnp.float32)]
    + [jax.ShapeDtypeStruct((N, W2), jnp.float32)] * 4,
    compiler_params=_TC_PARAMS,
)

_tc_post = pl.pallas_call(
    _tc_post_body,
    out_shape=[jax.ShapeDtypeStruct((G, 1), jnp.float32),
               jax.ShapeDtypeStruct((G, 1), jnp.float32)],
    compiler_params=_TC_PARAMS,
)


def kernel(x, pos, batch, edge_index, W_enc, b_enc,
           Wm0, bm0, Wu0, bu0, Wp0, bp0,
           Wm1, bm1, Wu1, bu1, Wp1, bp1,
           Wm2, bm2, Wu2, bu2, Wp2, bp2,
           Wg1, bg1, Wg2, bg2, Wf1, bf1, Wf2, bf2):
    f32 = jnp.float32
    row = edge_index[0]
    col = edge_index[1]

    wenc_p = jnp.concatenate([W_enc, jnp.zeros((D, D - H), f32)], axis=1)
    benc_p = jnp.concatenate([b_enc, jnp.zeros((D - H,), f32)])[None, :]
    wcat = jnp.concatenate(
        [Wm0[:H], Wm1[:H], Wm2[:H], Wm0[H:2 * H], Wm1[H:2 * H], Wm2[H:2 * H]],
        axis=1)
    wcat = jnp.concatenate([wcat, jnp.zeros((D - H, 2 * F), f32)], axis=0)
    wvec = jnp.concatenate([Wm0[2 * H], Wm1[2 * H], Wm2[2 * H]])
    wvec = wvec.astype(jnp.bfloat16).astype(f32)
    bvec = jnp.concatenate([bm0, bm1, bm2])[None, :]
    pad16 = jnp.zeros((W2 - FH,), f32)
    wvec2 = jnp.stack([jnp.concatenate([wvec[:FH], pad16]),
                       jnp.concatenate([wvec[FH:], pad16])])
    pos16 = jnp.concatenate([pos, jnp.zeros((N, 16 - 3), f32)], axis=1)
    zeros_tab = jnp.zeros((N, W2), f32)

    h_pad, td0, td1, ts0, ts1 = _tc_pre(x, wenc_p, benc_p, wcat, pos16, bvec)
    idx2 = jnp.stack([col.reshape(E // CH, CH), row.reshape(E // CH, CH)],
                     axis=1)
    aggr2 = _get_sc_edge()(td0, td1, ts0, ts1, idx2, wvec2, zeros_tab)

    wu = jnp.stack([Wu0, Wu1, Wu2])
    bu = jnp.stack([bu0, bu1, bu2])[:, None, :]
    wp = jnp.stack([Wp0, Wp1, Wp2])
    bp = jnp.stack([bp0, bp1, bp2])[:, None, :]
    geom, func = _tc_post(h_pad, aggr2, batch[None, :].astype(jnp.int32),
                          wu, bu, wp, bp,
                          Wg1, bg1[None, :], Wg2, bg2[None, :],
                          Wf1, bf1[None, :], Wf2, bf2[None, :])
    return (geom, func)
